# baseline (speedup 1.0000x reference)
import functools

import jax
import jax.numpy as jnp
from jax import lax
from jax.experimental import pallas as pl
from jax.experimental.pallas import tpu as pltpu
from jax.experimental.pallas import tpu_sc as plsc

D = 32
CHUNK = 128
NW = 32
NC = 2


def _rsqrt(t):
    i = lax.bitcast_convert_type(t, jnp.int32)
    i = jnp.int32(0x5F3759DF) - lax.shift_right_logical(i, 1)
    y = lax.bitcast_convert_type(i, jnp.float32)
    half = jnp.float32(0.5)
    three_half = jnp.float32(1.5)
    for _ in range(3):
        y = y * (three_half - half * t * y * y)
    return y


def _cosine_sc(tT, i1, i2, batch):
    n_chunks = i1.shape[1]
    per_w = n_chunks * CHUNK
    n_vregs = per_w // 16
    mesh = plsc.VectorSubcoreMesh(core_axis_name="c", subcore_axis_name="s")

    @functools.partial(
        pl.kernel,
        out_type=jax.ShapeDtypeStruct((batch,), jnp.float32),
        mesh=mesh,
        compiler_params=pltpu.CompilerParams(use_tc_tiling_on_sc=False),
        scratch_types=[
            pltpu.VMEM((n_chunks, CHUNK), jnp.int32),
            pltpu.VMEM((n_chunks, CHUNK), jnp.int32),
            pltpu.VMEM((D, per_w), jnp.float32),
            pltpu.VMEM((D, per_w), jnp.float32),
            pltpu.VMEM((per_w,), jnp.float32),
            pltpu.SemaphoreType.DMA,
        ],
    )
    def k(tT_hbm, i1_hbm, i2_hbm, out_hbm, i1_v, i2_v, g1_v, g2_v, out_v, sem):
        wid = lax.axis_index("s") * NC + lax.axis_index("c")
        base = wid * per_w
        pltpu.sync_copy(i1_hbm.at[wid], i1_v)
        pltpu.sync_copy(i2_hbm.at[wid], i2_v)
        handles = []
        for d in range(D):
            row = tT_hbm.at[d]
            for c in range(n_chunks):
                dst = pl.ds(c * CHUNK, CHUNK)
                handles.append(
                    pltpu.async_copy(row.at[i1_v.at[c]], g1_v.at[d, dst], sem)
                )
                handles.append(
                    pltpu.async_copy(row.at[i2_v.at[c]], g2_v.at[d, dst], sem)
                )
        for h in handles:
            h.wait()

        def body(j, _):
            sl = pl.ds(j * 16, 16)
            dot = jnp.zeros((16,), jnp.float32)
            s1 = jnp.zeros((16,), jnp.float32)
            s2 = jnp.zeros((16,), jnp.float32)
            for d in range(D):
                a = g1_v[d, sl]
                b = g2_v[d, sl]
                dot += a * b
                s1 += a * a
                s2 += b * b
            eps2 = jnp.float32(1e-16)
            t = jnp.maximum(s1, eps2) * jnp.maximum(s2, eps2)
            sim = dot * _rsqrt(t)
            out_v[sl] = jnp.float32(0.5) + jnp.float32(0.5) * sim
            return 0

        lax.fori_loop(0, n_vregs, body, 0)
        pltpu.sync_copy(out_v, out_hbm.at[pl.ds(base, per_w)])

    return k(tT, i1, i2)


def kernel(x, table):
    x = x.reshape(-1, 2)
    batch = x.shape[0]
    n_chunks = batch // (NW * CHUNK)
    tT = table.T
    i1 = x[:, 0].astype(jnp.int32).reshape(NW, n_chunks, CHUNK)
    i2 = x[:, 1].astype(jnp.int32).reshape(NW, n_chunks, CHUNK)
    return _cosine_sc(tT, i1, i2, batch)


# speedup vs baseline: 4.7491x; 4.7491x over previous
import functools

import jax
import jax.numpy as jnp
from jax import lax
from jax.experimental import pallas as pl
from jax.experimental.pallas import tpu as pltpu
from jax.experimental.pallas import tpu_sc as plsc

D = 32
CHUNK = 128


def _gather_sc(table, idx3, n_chunks):
    NW = idx3.shape[0]
    NC = 2
    per_w = n_chunks * CHUNK
    R = NW * per_w
    mesh = plsc.VectorSubcoreMesh(core_axis_name="c", subcore_axis_name="s")

    @functools.partial(
        pl.kernel,
        out_type=jax.ShapeDtypeStruct((R, D), jnp.float32),
        mesh=mesh,
        compiler_params=pltpu.CompilerParams(use_tc_tiling_on_sc=False),
        scratch_types=[
            pltpu.VMEM((n_chunks, CHUNK), jnp.int32),
            pltpu.VMEM((per_w, D), jnp.float32),
            pltpu.SemaphoreType.DMA,
        ],
    )
    def k(table_hbm, idx_hbm, out_hbm, idx_v, rows_v, sem):
        wid = lax.axis_index("s") * NC + lax.axis_index("c")
        base = wid * per_w
        pltpu.sync_copy(idx_hbm.at[wid], idx_v)
        handles = []
        for c in range(n_chunks):
            handles.append(
                pltpu.async_copy(
                    table_hbm.at[idx_v.at[c]],
                    rows_v.at[pl.ds(c * CHUNK, CHUNK)],
                    sem,
                )
            )
        for h in handles:
            h.wait()
        pltpu.sync_copy(rows_v, out_hbm.at[pl.ds(base, per_w)])

    return k(table, idx3)


def _cosine_tc(rows, batch):
    def body(r_ref, o_ref):
        r = r_ref[...]
        e1 = r[:, :D]
        e2 = r[:, D:]
        dot = jnp.sum(e1 * e2, axis=1)
        s1 = jnp.sum(e1 * e1, axis=1)
        s2 = jnp.sum(e2 * e2, axis=1)
        eps = jnp.float32(1e-8)
        n1 = jnp.maximum(jnp.sqrt(s1), eps)
        n2 = jnp.maximum(jnp.sqrt(s2), eps)
        o_ref[...] = 0.5 + 0.5 * (dot / (n1 * n2))

    return pl.pallas_call(
        body,
        out_shape=jax.ShapeDtypeStruct((batch,), jnp.float32),
    )(rows)


def kernel(x, table):
    x = x.reshape(-1, 2)
    batch = x.shape[0]
    idx_flat = x.reshape(-1).astype(jnp.int32)
    NW = 32
    n_chunks = (2 * batch) // (NW * CHUNK)
    idx3 = idx_flat.reshape(NW, n_chunks, CHUNK)
    rows = _gather_sc(table, idx3, n_chunks)
    rows2 = rows.reshape(batch, 2 * D)
    return _cosine_tc(rows2, batch)


# speedup vs baseline: 7.9683x; 1.6779x over previous
import functools

import jax
import jax.numpy as jnp
from jax import lax
from jax.experimental import pallas as pl
from jax.experimental.pallas import tpu as pltpu
from jax.experimental.pallas import tpu_sc as plsc

D = 32
NW = 32
NC = 2


def _rsqrt(t):
    i = lax.bitcast_convert_type(t, jnp.int32)
    i = jnp.int32(0x5F3759DF) - lax.shift_right_logical(i, 1)
    y = lax.bitcast_convert_type(i, jnp.float32)
    half = jnp.float32(0.5)
    three_half = jnp.float32(1.5)
    for _ in range(3):
        y = y * (three_half - half * t * y * y)
    return y


def _fused_sc(table, idx, batch):
    per_w = batch // NW
    slots = 2 * per_w
    groups = slots // 16
    vrows = slots // 4
    mesh = plsc.VectorSubcoreMesh(core_axis_name="c", subcore_axis_name="s")

    @functools.partial(
        pl.kernel,
        out_type=jax.ShapeDtypeStruct((batch,), jnp.float32),
        mesh=mesh,
        compiler_params=pltpu.CompilerParams(
            use_tc_tiling_on_sc=True, needs_layout_passes=False
        ),
        scratch_types=[
            pltpu.VMEM((slots // 128, 128), jnp.int32),
            pltpu.VMEM((vrows, 128), jnp.float32),
            pltpu.VMEM((per_w,), jnp.float32),
            pltpu.SemaphoreType.DMA,
        ],
    )
    def k(table_hbm, idx_hbm, out_hbm, idx_v, rows_v, out_v, sem):
        wid = lax.axis_index("s") * NC + lax.axis_index("c")
        pltpu.sync_copy(idx_hbm.at[wid], idx_v)

        def fire(g, _):
            ivec = idx_v[g // 8, pl.ds((g % 8) * 16, 16)]
            for l in range(16):
                pltpu.async_copy(
                    table_hbm.at[ivec[l]],
                    rows_v.at[g * 4 + l // 4, pl.ds((l % 4) * 32, 32)],
                    sem,
                )
            return 0

        lax.fori_loop(0, groups, fire, 0)

        def drain(j, _):
            pltpu.make_async_copy(
                table_hbm.at[0], rows_v.at[0, pl.ds(0, 32)], sem
            ).wait()
            return 0

        lax.fori_loop(0, slots, drain, 0)

        def comp(g, _):
            lanes = lax.iota(jnp.int32, 16)
            onehots = [lanes == jnp.int32(l) for l in range(16)]
            dot = jnp.zeros((16,), jnp.float32)
            s1 = jnp.zeros((16,), jnp.float32)
            s2 = jnp.zeros((16,), jnp.float32)
            for l in range(16):
                row = g * 8 + l // 2
                col = (l % 2) * 64
                a0 = rows_v[row, pl.ds(col, 16)]
                a1 = rows_v[row, pl.ds(col + 16, 16)]
                b0 = rows_v[row, pl.ds(col + 32, 16)]
                b1 = rows_v[row, pl.ds(col + 48, 16)]
                dot = jnp.where(onehots[l], jnp.sum(a0 * b0 + a1 * b1), dot)
                s1 = jnp.where(onehots[l], jnp.sum(a0 * a0 + a1 * a1), s1)
                s2 = jnp.where(onehots[l], jnp.sum(b0 * b0 + b1 * b1), s2)
            eps2 = jnp.float32(1e-16)
            t = jnp.maximum(s1, eps2) * jnp.maximum(s2, eps2)
            sim = dot * _rsqrt(t)
            out_v[pl.ds(g * 16, 16)] = jnp.float32(0.5) + jnp.float32(0.5) * sim
            return 0

        lax.fori_loop(0, per_w // 16, comp, 0)
        base = pl.multiple_of(wid * per_w, 8)
        pltpu.sync_copy(out_v, out_hbm.at[pl.ds(base, per_w)])

    return k(table, idx)


def kernel(x, table):
    x = x.reshape(-1, 2)
    batch = x.shape[0]
    slots = (2 * batch) // NW
    idx = x.astype(jnp.int32).reshape(NW, slots // 128, 128)
    return _fused_sc(table, idx, batch)
